# baseline (device time: 85987 ns/iter reference)
import jax
import jax.numpy as jnp
from jax import lax
from jax.experimental import pallas as pl
from jax.experimental.pallas import tpu as pltpu

N_DEV = 4


def kernel(x, w_mat):
    m, k_per = x.shape
    _, n = w_mat.shape
    mc = m // N_DEV

    def body(x_ref, w_ref, out_ref, comm_ref, send_sems, recv_sems):
        my = lax.axis_index("i")
        left = lax.rem(my + N_DEV - 1, N_DEV)
        right = lax.rem(my + 1, N_DEV)

        barrier_sem = pltpu.get_barrier_semaphore()
        for nbr in (left, right):
            pl.semaphore_signal(
                barrier_sem, inc=1,
                device_id=(nbr,), device_id_type=pl.DeviceIdType.MESH,
            )
        pl.semaphore_wait(barrier_sem, 2)

        out_ref[...] = jnp.dot(
            x_ref[...], w_ref[...], preferred_element_type=jnp.float32
        )

        for s in range(N_DEV - 1):
            send_c = lax.rem(my - s + N_DEV, N_DEV)
            rdma = pltpu.make_async_remote_copy(
                src_ref=out_ref.at[pl.ds(send_c * mc, mc), :],
                dst_ref=comm_ref.at[s],
                send_sem=send_sems.at[s],
                recv_sem=recv_sems.at[s],
                device_id=(right,),
                device_id_type=pl.DeviceIdType.MESH,
            )
            rdma.start()
            rdma.wait()
            recv_c = lax.rem(my - s - 1 + N_DEV, N_DEV)
            sl = pl.ds(recv_c * mc, mc)
            out_ref[sl, :] = out_ref[sl, :] + comm_ref[s]

        own = lax.rem(my + 1, N_DEV)
        sl = pl.ds(own * mc, mc)
        y = out_ref[sl, :]
        out_ref[sl, :] = y * jax.nn.sigmoid(y)

        for s in range(N_DEV - 1):
            send_c = lax.rem(my + 1 - s + N_DEV, N_DEV)
            rdma = pltpu.make_async_remote_copy(
                src_ref=out_ref.at[pl.ds(send_c * mc, mc), :],
                dst_ref=comm_ref.at[N_DEV - 1 + s],
                send_sem=send_sems.at[N_DEV - 1 + s],
                recv_sem=recv_sems.at[N_DEV - 1 + s],
                device_id=(right,),
                device_id_type=pl.DeviceIdType.MESH,
            )
            rdma.start()
            rdma.wait()
            recv_c = lax.rem(my - s + N_DEV, N_DEV)
            out_ref[pl.ds(recv_c * mc, mc), :] = comm_ref[N_DEV - 1 + s]

    n_hops = 2 * (N_DEV - 1)
    return pl.pallas_call(
        body,
        out_shape=jax.ShapeDtypeStruct((m, n), jnp.float32),
        in_specs=[
            pl.BlockSpec(memory_space=pltpu.VMEM),
            pl.BlockSpec(memory_space=pltpu.VMEM),
        ],
        out_specs=pl.BlockSpec(memory_space=pltpu.VMEM),
        scratch_shapes=[
            pltpu.VMEM((n_hops, mc, n), jnp.float32),
            pltpu.SemaphoreType.DMA((n_hops,)),
            pltpu.SemaphoreType.DMA((n_hops,)),
        ],
        compiler_params=pltpu.CompilerParams(collective_id=0),
    )(x, w_mat)


# device time: 52832 ns/iter; 1.6276x vs baseline; 1.6276x over previous
import jax
import jax.numpy as jnp
from jax import lax
from jax.experimental import pallas as pl
from jax.experimental.pallas import tpu as pltpu

N_DEV = 4


def kernel(x, w_mat):
    m, k_per = x.shape
    _, n = w_mat.shape
    mc = m // N_DEV
    half = n // 2

    def body(x_ref, w_ref, out_ref,
             comm_r, comm_l, send_r, recv_r, send_l, recv_l):
        my = lax.axis_index("i")
        left = lax.rem(my + N_DEV - 1, N_DEV)
        right = lax.rem(my + 1, N_DEV)
        cols_r = pl.ds(0, half)
        cols_l = pl.ds(half, half)

        barrier_sem = pltpu.get_barrier_semaphore()
        for nbr in (left, right):
            pl.semaphore_signal(
                barrier_sem, inc=1,
                device_id=(nbr,), device_id_type=pl.DeviceIdType.MESH,
            )
        pl.semaphore_wait(barrier_sem, 2)

        out_ref[...] = jnp.dot(
            x_ref[...], w_ref[...], preferred_element_type=jnp.float32
        )

        def hop(s, src_chunk_r, src_chunk_l):
            rdma_r = pltpu.make_async_remote_copy(
                src_ref=out_ref.at[pl.ds(src_chunk_r * mc, mc), cols_r],
                dst_ref=comm_r.at[s],
                send_sem=send_r.at[s],
                recv_sem=recv_r.at[s],
                device_id=(right,),
                device_id_type=pl.DeviceIdType.MESH,
            )
            rdma_l = pltpu.make_async_remote_copy(
                src_ref=out_ref.at[pl.ds(src_chunk_l * mc, mc), cols_l],
                dst_ref=comm_l.at[s],
                send_sem=send_l.at[s],
                recv_sem=recv_l.at[s],
                device_id=(left,),
                device_id_type=pl.DeviceIdType.MESH,
            )
            rdma_r.start()
            rdma_l.start()
            rdma_r.wait()
            rdma_l.wait()

        for s in range(N_DEV - 1):
            hop(s,
                lax.rem(my - s + N_DEV, N_DEV),
                lax.rem(my + s, N_DEV))
            rc_r = lax.rem(my - s - 1 + N_DEV, N_DEV)
            rc_l = lax.rem(my + s + 1, N_DEV)
            sl_r = pl.ds(rc_r * mc, mc)
            sl_l = pl.ds(rc_l * mc, mc)
            out_ref[sl_r, cols_r] = out_ref[sl_r, cols_r] + comm_r[s]
            out_ref[sl_l, cols_l] = out_ref[sl_l, cols_l] + comm_l[s]

        own_r = lax.rem(my + 1, N_DEV)
        own_l = lax.rem(my + N_DEV - 1, N_DEV)
        sl = pl.ds(own_r * mc, mc)
        y = out_ref[sl, cols_r]
        out_ref[sl, cols_r] = y * jax.nn.sigmoid(y)
        sl = pl.ds(own_l * mc, mc)
        y = out_ref[sl, cols_l]
        out_ref[sl, cols_l] = y * jax.nn.sigmoid(y)

        for s in range(N_DEV - 1):
            hop(N_DEV - 1 + s,
                lax.rem(my + 1 - s + N_DEV, N_DEV),
                lax.rem(my - 1 + s + N_DEV, N_DEV))
            rc_r = lax.rem(my - s + N_DEV, N_DEV)
            rc_l = lax.rem(my + s, N_DEV)
            out_ref[pl.ds(rc_r * mc, mc), cols_r] = comm_r[N_DEV - 1 + s]
            out_ref[pl.ds(rc_l * mc, mc), cols_l] = comm_l[N_DEV - 1 + s]

    n_hops = 2 * (N_DEV - 1)
    return pl.pallas_call(
        body,
        out_shape=jax.ShapeDtypeStruct((m, n), jnp.float32),
        in_specs=[
            pl.BlockSpec(memory_space=pltpu.VMEM),
            pl.BlockSpec(memory_space=pltpu.VMEM),
        ],
        out_specs=pl.BlockSpec(memory_space=pltpu.VMEM),
        scratch_shapes=[
            pltpu.VMEM((n_hops, mc, half), jnp.float32),
            pltpu.VMEM((n_hops, mc, half), jnp.float32),
            pltpu.SemaphoreType.DMA((n_hops,)),
            pltpu.SemaphoreType.DMA((n_hops,)),
            pltpu.SemaphoreType.DMA((n_hops,)),
            pltpu.SemaphoreType.DMA((n_hops,)),
        ],
        compiler_params=pltpu.CompilerParams(collective_id=0),
    )(x, w_mat)


# device time: 43754 ns/iter; 1.9652x vs baseline; 1.2075x over previous
import jax
import jax.numpy as jnp
from jax import lax
from jax.experimental import pallas as pl
from jax.experimental.pallas import tpu as pltpu

N_DEV = 4
NB = 2
N_SLOTS = 2 * (N_DEV - 1)


def kernel(x, w_mat):
    m, k_per = x.shape
    _, n = w_mat.shape
    mc = m // N_DEV
    mcb = mc // NB
    half = n // 2

    def body(x_ref, w_ref, out_ref,
             comm_r, comm_l, send_r, recv_r, send_l, recv_l):
        my = lax.axis_index("i")
        left = lax.rem(my + N_DEV - 1, N_DEV)
        right = lax.rem(my + 1, N_DEV)
        cols = {"r": pl.ds(0, half), "l": pl.ds(half, half)}
        comm = {"r": comm_r, "l": comm_l}
        ssem = {"r": send_r, "l": send_l}
        rsem = {"r": recv_r, "l": recv_l}
        peer = {"r": right, "l": left}
        sgn = {"r": -1, "l": +1}
        descs = {}

        def chunk_rows(c, b):
            return pl.ds(c * mc + b * mcb, mcb)

        def start(d, slot, b, src_ref):
            r = pltpu.make_async_remote_copy(
                src_ref=src_ref,
                dst_ref=comm[d].at[slot, pl.ds(b * mcb, mcb), :],
                send_sem=ssem[d].at[slot * NB + b],
                recv_sem=rsem[d].at[slot * NB + b],
                device_id=(peer[d],),
                device_id_type=pl.DeviceIdType.MESH,
            )
            r.start()
            descs[d, slot, b] = r

        barrier_sem = pltpu.get_barrier_semaphore()
        for nbr in (left, right):
            pl.semaphore_signal(
                barrier_sem, inc=1,
                device_id=(nbr,), device_id_type=pl.DeviceIdType.MESH,
            )
        pl.semaphore_wait(barrier_sem, 2)

        def gemm_chunk(c):
            sl = pl.ds(c * mc, mc)
            out_ref[sl, :] = jnp.dot(
                x_ref[sl, :], w_ref[...], preferred_element_type=jnp.float32
            )

        gemm_chunk(my)
        for b in range(NB):
            for d in ("r", "l"):
                start(d, 0, b, out_ref.at[chunk_rows(my, b), cols[d]])
        for k in range(1, N_DEV):
            gemm_chunk(lax.rem(my + k, N_DEV))

        for s in range(1, N_DEV - 1):
            for b in range(NB):
                for d in ("r", "l"):
                    c = lax.rem(my + sgn[d] * s + N_DEV, N_DEV)
                    rows = chunk_rows(c, b)
                    descs[d, s - 1, b].wait_recv()
                    out_ref[rows, cols[d]] = (
                        out_ref[rows, cols[d]]
                        + comm[d][s - 1, pl.ds(b * mcb, mcb), :]
                    )
                    start(d, s, b, out_ref.at[rows, cols[d]])

        for b in range(NB):
            for d in ("r", "l"):
                own = lax.rem(my - sgn[d] + N_DEV, N_DEV)
                rows = chunk_rows(own, b)
                descs[d, N_DEV - 2, b].wait_recv()
                y = out_ref[rows, cols[d]] + comm[d][
                    N_DEV - 2, pl.ds(b * mcb, mcb), :
                ]
                out_ref[rows, cols[d]] = y * jax.nn.sigmoid(y)
                start(d, N_DEV - 1, b, out_ref.at[rows, cols[d]])

        for t in range(N_DEV - 1):
            ag = N_DEV - 1 + t
            for b in range(NB):
                for d in ("r", "l"):
                    c = lax.rem(my + sgn[d] * t + N_DEV, N_DEV)
                    descs[d, ag, b].wait_recv()
                    if t < N_DEV - 2:
                        start(d, ag + 1, b,
                              comm[d].at[ag, pl.ds(b * mcb, mcb), :])
                    descs[d, t, b].wait_send()
                    out_ref[chunk_rows(c, b), cols[d]] = comm[d][
                        ag, pl.ds(b * mcb, mcb), :
                    ]

        for t in range(N_DEV - 1):
            for b in range(NB):
                for d in ("r", "l"):
                    descs[d, N_DEV - 1 + t, b].wait_send()

    return pl.pallas_call(
        body,
        out_shape=jax.ShapeDtypeStruct((m, n), jnp.float32),
        in_specs=[
            pl.BlockSpec(memory_space=pltpu.VMEM),
            pl.BlockSpec(memory_space=pltpu.VMEM),
        ],
        out_specs=pl.BlockSpec(memory_space=pltpu.VMEM),
        scratch_shapes=[
            pltpu.VMEM((N_SLOTS, mc, half), jnp.float32),
            pltpu.VMEM((N_SLOTS, mc, half), jnp.float32),
            pltpu.SemaphoreType.DMA((N_SLOTS * NB,)),
            pltpu.SemaphoreType.DMA((N_SLOTS * NB,)),
            pltpu.SemaphoreType.DMA((N_SLOTS * NB,)),
            pltpu.SemaphoreType.DMA((N_SLOTS * NB,)),
        ],
        compiler_params=pltpu.CompilerParams(collective_id=0),
    )(x, w_mat)
